# Initial kernel scaffold; baseline (speedup 1.0000x reference)
import jax
import jax.numpy as jnp
from jax.experimental import pallas as pl


def kernel(queries, keys, labels):
    raise NotImplementedError("write your pallas kernel here")


# speedup vs baseline: 26.3195x; 26.3195x over previous
import functools

import jax
import jax.numpy as jnp
from jax import lax
from jax.experimental import pallas as pl
from jax.experimental.pallas import tpu as pltpu
from jax.experimental.pallas import tpu_sc as plsc

NKEY = 100000
NKP = 102400
CQ = 256
CK = 4096
NCHUNK = NKP // CK
W = 128
NB = NKP // W
TSEL = 24
NCAND = TSEL * W
NEGF = -1e30
BIGI = 2**30

def _sim_kernel(q_ref, k_ref, sim_ref, bid_ref, macc_ref):
    j = pl.program_id(1)
    sim = lax.dot_general(
        q_ref[...], k_ref[...], (((1,), (1,)), ((), ())),
        preferred_element_type=jnp.float32,
        precision=lax.Precision.DEFAULT,
    )
    col = j * CK + lax.broadcasted_iota(jnp.int32, (CQ, CK), 1)
    sim = jnp.where(col < NKEY, sim, NEGF)
    sim_ref[...] = sim
    macc_ref[j] = jnp.max(sim.reshape(CQ, CK // W, W), axis=2)

    @pl.when(j == NCHUNK - 1)
    def _extract():
        x3 = macc_ref[...]
        cidx = lax.broadcasted_iota(jnp.int32, (NCHUNK, CQ, 32), 0)
        lidx = lax.broadcasted_iota(jnp.int32, (NCHUNK, CQ, 32), 2)
        b3 = cidx * 32 + lidx
        lane32 = lax.broadcasted_iota(jnp.int32, (CQ, 32), 1)
        bacc = jnp.zeros((CQ, 32), jnp.int32)
        for t in range(TSEL):
            m1 = jnp.max(x3, axis=0)
            mx = jnp.max(m1, axis=1, keepdims=True)
            pb = jnp.where(x3 == mx[None, :, :], b3, BIGI)
            p1 = jnp.min(pb, axis=0)
            p = jnp.min(p1, axis=1, keepdims=True)
            bacc = jnp.where(lane32 == t, p, bacc)
            x3 = jnp.where(b3 == p[None, :, :], NEGF, x3)
        bid_ref[...] = bacc


def _final_kernel(c_ref, gi_ref, lab_ref, rnd_ref, v_ref, i_ref, fi_ref):
    x = c_ref[...]
    gi = gi_ref[...]
    x = jnp.where(gi == lab_ref[...], -1e9, x)
    gs = []
    for t in range(20):
        mx = jnp.max(x, axis=1, keepdims=True)
        g = jnp.min(jnp.where(x == mx, gi, BIGI), axis=1, keepdims=True)
        v_ref[:, t:t + 1] = mx
        i_ref[:, t:t + 1] = g
        gs.append(g)
        x = jnp.where(gi == g, NEGF, x)
    rnd = rnd_ref[...]
    i20 = lax.broadcasted_iota(jnp.int32, (CQ, 20), 1)
    rmx = jnp.max(rnd, axis=1, keepdims=True)
    p = jnp.min(jnp.where(rnd == rmx, i20, BIGI), axis=1, keepdims=True)
    fi = gs[0]
    for t in range(1, 20):
        fi = jnp.where(p == t, gs[t], fi)
    fi_ref[...] = fi


def _sc_gather(table, gidx2d, nrows):
    info = plsc.get_sparse_core_info()
    nc, ns = info.num_cores, info.num_subcores
    nw = nc * ns
    per_w = nrows // nw
    nchips = per_w // 128
    mesh = plsc.VectorSubcoreMesh(core_axis_name="c", subcore_axis_name="s")

    @functools.partial(
        pl.kernel,
        out_type=jax.ShapeDtypeStruct((nrows, W), jnp.float32),
        mesh=mesh,
        scratch_types=[
            pltpu.VMEM((8, 128), jnp.int32),
            pltpu.VMEM((per_w, W), jnp.float32),
            pltpu.SemaphoreType.DMA,
        ],
    )
    def k(table_hbm, idx_hbm, out_hbm, idx_v, rows_v, sem):
        wid = lax.axis_index("s") * nc + lax.axis_index("c")
        pltpu.sync_copy(idx_hbm.at[pl.ds(wid * 8, 8)], idx_v)
        copies = [
            pltpu.async_copy(table_hbm.at[idx_v.at[j]],
                             rows_v.at[pl.ds(j * 128, 128)], sem)
            for j in range(nchips)
        ]
        for c in copies:
            c.wait()
        pltpu.sync_copy(rows_v, out_hbm.at[pl.ds(wid * per_w, per_w)])

    return k(table, gidx2d)


def kernel(queries, keys, labels):
    Q = queries.shape[0]
    keys_p = jnp.pad(keys, ((0, NKP - NKEY), (0, 0)))

    sim, bid = pl.pallas_call(
        _sim_kernel,
        grid=(Q // CQ, NCHUNK),
        in_specs=[
            pl.BlockSpec((CQ, 128), lambda i, j: (i, 0)),
            pl.BlockSpec((CK, 128), lambda i, j: (j, 0)),
        ],
        out_specs=[
            pl.BlockSpec((CQ, CK), lambda i, j: (i, j)),
            pl.BlockSpec((CQ, 32), lambda i, j: (i, 0)),
        ],
        out_shape=[
            jax.ShapeDtypeStruct((Q, NKP), jnp.float32),
            jax.ShapeDtypeStruct((Q, 32), jnp.int32),
        ],
        scratch_shapes=[pltpu.VMEM((NCHUNK, CQ, 32), jnp.float32)],
    )(queries, keys_p)

    bid24 = bid[:, :TSEL]
    qi = jnp.arange(Q, dtype=jnp.int32)[:, None]
    gidx = (qi * NB + bid24).reshape(32, Q * TSEL // (128 * 32), 128)
    gidx = jnp.pad(gidx, ((0, 0), (0, 8 - gidx.shape[1]), (0, 0)))
    table = sim.reshape(Q * NB, W)
    cand = _sc_gather(table, gidx.reshape(32 * 8, 128), Q * TSEL)

    cand2 = cand.reshape(Q, NCAND)
    gi = (bid24[:, :, None] * W
          + jnp.arange(W, dtype=jnp.int32)).reshape(Q, NCAND)
    rnd = jax.random.uniform(jax.random.key(1), (Q, 20), dtype=jnp.float32)

    vals2, idx2, fi = pl.pallas_call(
        _final_kernel,
        grid=(Q // CQ,),
        in_specs=[
            pl.BlockSpec((CQ, NCAND), lambda i: (i, 0)),
            pl.BlockSpec((CQ, NCAND), lambda i: (i, 0)),
            pl.BlockSpec((CQ, 1), lambda i: (i, 0)),
            pl.BlockSpec((CQ, 20), lambda i: (i, 0)),
        ],
        out_specs=[
            pl.BlockSpec((CQ, 20), lambda i: (i, 0)),
            pl.BlockSpec((CQ, 20), lambda i: (i, 0)),
            pl.BlockSpec((CQ, 1), lambda i: (i, 0)),
        ],
        out_shape=[
            jax.ShapeDtypeStruct((Q, 20), jnp.float32),
            jax.ShapeDtypeStruct((Q, 20), jnp.int32),
            jax.ShapeDtypeStruct((Q, 1), jnp.int32),
        ],
    )(cand2, gi, labels.reshape(Q, 1), rnd)

    return vals2, idx2, fi


# speedup vs baseline: 31.0876x; 1.1812x over previous
import functools

import jax
import jax.numpy as jnp
from jax import lax
from jax.experimental import pallas as pl
from jax.experimental.pallas import tpu as pltpu
from jax.experimental.pallas import tpu_sc as plsc

NKEY = 100000
NKP = 102400
CQ = 256
CK = 4096
NCHUNK = NKP // CK
W = 128
NB = NKP // W
TSEL = 24
NCAND = TSEL * W
NEGF = -1e30
BIGI = 2**30

def _sim_kernel(q_ref, k_ref, sim_ref, bid_ref, macc_ref):
    j = pl.program_id(1)
    sim = lax.dot_general(
        q_ref[...], k_ref[...], (((1,), (1,)), ((), ())),
        preferred_element_type=jnp.float32,
        precision=lax.Precision.DEFAULT,
    )
    col = j * CK + lax.broadcasted_iota(jnp.int32, (CQ, CK), 1)
    sim = jnp.where(col < NKEY, sim, NEGF)
    sim3 = sim.reshape(CQ, CK // W, W)
    sim_ref[...] = sim3
    macc_ref[j] = jnp.max(sim3, axis=2)

    @pl.when(j == NCHUNK - 1)
    def _extract():
        x3 = macc_ref[...]
        cidx = lax.broadcasted_iota(jnp.int32, (NCHUNK, CQ, 32), 0)
        lidx = lax.broadcasted_iota(jnp.int32, (NCHUNK, CQ, 32), 2)
        b3 = cidx * 32 + lidx
        lane32 = lax.broadcasted_iota(jnp.int32, (CQ, 32), 1)
        bacc = jnp.zeros((CQ, 32), jnp.int32)
        for t in range(TSEL):
            m1 = jnp.max(x3, axis=0)
            mx = jnp.max(m1, axis=1, keepdims=True)
            pb = jnp.where(x3 == mx[None, :, :], b3, BIGI)
            p1 = jnp.min(pb, axis=0)
            p = jnp.min(p1, axis=1, keepdims=True)
            bacc = jnp.where(lane32 == t, p, bacc)
            x3 = jnp.where(b3 == p[None, :, :], NEGF, x3)
        bid_ref[...] = bacc


def _final_kernel(c_ref, bid_ref, lab_ref, rnd_ref, v_ref, i_ref, fi_ref):
    x = c_ref[...]
    bid = bid_ref[...][:, :TSEL]
    gi = (bid[:, :, None] * W
          + lax.broadcasted_iota(jnp.int32, (CQ, TSEL, W), 2))
    lab = lab_ref[...][:, :, None]
    x = jnp.where(gi == lab, -1e9, x)
    gs = []
    for t in range(20):
        m1 = jnp.max(x, axis=2)
        mx = jnp.max(m1, axis=1, keepdims=True)
        g1 = jnp.min(jnp.where(x == mx[:, :, None], gi, BIGI), axis=2)
        g = jnp.min(g1, axis=1, keepdims=True)
        v_ref[:, t:t + 1] = mx
        i_ref[:, t:t + 1] = g
        gs.append(g)
        x = jnp.where(gi == g[:, :, None], NEGF, x)
    rnd = rnd_ref[...]
    i20 = lax.broadcasted_iota(jnp.int32, (CQ, 20), 1)
    rmx = jnp.max(rnd, axis=1, keepdims=True)
    p = jnp.min(jnp.where(rnd == rmx, i20, BIGI), axis=1, keepdims=True)
    fi = gs[0]
    for t in range(1, 20):
        fi = jnp.where(p == t, gs[t], fi)
    fi_ref[...] = fi


def _sc_gather(table, gidx2d, nrows):
    info = plsc.get_sparse_core_info()
    nc, ns = info.num_cores, info.num_subcores
    nw = nc * ns
    per_w = nrows // nw
    nchips = per_w // 128
    mesh = plsc.VectorSubcoreMesh(core_axis_name="c", subcore_axis_name="s")

    @functools.partial(
        pl.kernel,
        out_type=jax.ShapeDtypeStruct((nrows, W), jnp.float32),
        mesh=mesh,
        scratch_types=[
            pltpu.VMEM((8, 128), jnp.int32),
            pltpu.VMEM((per_w, W), jnp.float32),
            pltpu.SemaphoreType.DMA,
        ],
    )
    def k(table_hbm, idx_hbm, out_hbm, idx_v, rows_v, sem):
        wid = lax.axis_index("s") * nc + lax.axis_index("c")
        pltpu.sync_copy(idx_hbm.at[pl.ds(wid * 8, 8)], idx_v)
        copies = [
            pltpu.async_copy(table_hbm.at[idx_v.at[j]],
                             rows_v.at[pl.ds(j * 128, 128)], sem)
            for j in range(nchips)
        ]
        for c in copies:
            c.wait()
        pltpu.sync_copy(rows_v, out_hbm.at[pl.ds(wid * per_w, per_w)])

    return k(table, gidx2d)


def kernel(queries, keys, labels):
    Q = queries.shape[0]
    keys_p = jnp.pad(keys, ((0, NKP - NKEY), (0, 0)))

    sim, bid = pl.pallas_call(
        _sim_kernel,
        grid=(Q // CQ, NCHUNK),
        in_specs=[
            pl.BlockSpec((CQ, 128), lambda i, j: (i, 0)),
            pl.BlockSpec((CK, 128), lambda i, j: (j, 0)),
        ],
        out_specs=[
            pl.BlockSpec((CQ, CK // W, W), lambda i, j: (i, j, 0)),
            pl.BlockSpec((CQ, 32), lambda i, j: (i, 0)),
        ],
        out_shape=[
            jax.ShapeDtypeStruct((Q, NB, W), jnp.float32),
            jax.ShapeDtypeStruct((Q, 32), jnp.int32),
        ],
        scratch_shapes=[pltpu.VMEM((NCHUNK, CQ, 32), jnp.float32)],
    )(queries, keys_p)

    bid24 = bid[:, :TSEL]
    qi = jnp.arange(Q, dtype=jnp.int32)[:, None]
    gidx = (qi * NB + bid24).reshape(32, Q * TSEL // (128 * 32), 128)
    gidx = jnp.pad(gidx, ((0, 0), (0, 8 - gidx.shape[1]), (0, 0)))
    table = sim.reshape(Q * NB, W)
    cand = _sc_gather(table, gidx.reshape(32 * 8, 128), Q * TSEL)

    cand3 = cand.reshape(Q, TSEL, W)
    rnd = jax.random.uniform(jax.random.key(1), (Q, 20), dtype=jnp.float32)

    vals2, idx2, fi = pl.pallas_call(
        _final_kernel,
        grid=(Q // CQ,),
        in_specs=[
            pl.BlockSpec((CQ, TSEL, W), lambda i: (i, 0, 0)),
            pl.BlockSpec((CQ, 32), lambda i: (i, 0)),
            pl.BlockSpec((CQ, 1), lambda i: (i, 0)),
            pl.BlockSpec((CQ, 20), lambda i: (i, 0)),
        ],
        out_specs=[
            pl.BlockSpec((CQ, 20), lambda i: (i, 0)),
            pl.BlockSpec((CQ, 20), lambda i: (i, 0)),
            pl.BlockSpec((CQ, 1), lambda i: (i, 0)),
        ],
        out_shape=[
            jax.ShapeDtypeStruct((Q, 20), jnp.float32),
            jax.ShapeDtypeStruct((Q, 20), jnp.int32),
            jax.ShapeDtypeStruct((Q, 1), jnp.int32),
        ],
    )(cand3, bid, labels.reshape(Q, 1), rnd)

    return vals2, idx2, fi


# speedup vs baseline: 42.1322x; 1.3553x over previous
import functools

import jax
import jax.numpy as jnp
from jax import lax
from jax.experimental import pallas as pl
from jax.experimental.pallas import tpu as pltpu
from jax.experimental.pallas import tpu_sc as plsc

NKEY = 100000
NKP = 102400
CQ = 256
CK = 4096
NCHUNK = NKP // CK
W = 128
NB = NKP // W
TSEL = 24
NCAND = TSEL * W
NEGF = -1e30
BIGI = 2**30

def _sim_kernel(q_ref, k_ref, sim_ref, bid_ref, macc_ref):
    j = pl.program_id(1)
    sim = lax.dot_general(
        q_ref[...], k_ref[...], (((1,), (1,)), ((), ())),
        preferred_element_type=jnp.float32,
        precision=lax.Precision.DEFAULT,
    )
    col = j * CK + lax.broadcasted_iota(jnp.int32, (CQ, CK), 1)
    sim = jnp.where(col < NKEY, sim, NEGF)
    sim3 = sim.reshape(CQ, CK // W, W)
    sim_ref[...] = sim3
    macc_ref[j] = jnp.max(sim3, axis=2)

    @pl.when(j == NCHUNK - 1)
    def _extract():
        x3 = macc_ref[...]
        cidx = lax.broadcasted_iota(jnp.int32, (NCHUNK, CQ, 32), 0)
        lidx = lax.broadcasted_iota(jnp.int32, (NCHUNK, CQ, 32), 2)
        b3 = cidx * 32 + lidx
        lane32 = lax.broadcasted_iota(jnp.int32, (CQ, 32), 1)
        bacc = jnp.zeros((CQ, 32), jnp.int32)
        for t in range(TSEL):
            m1 = jnp.max(x3, axis=0)
            mx = jnp.max(m1, axis=1, keepdims=True)
            pb = jnp.where(x3 == mx[None, :, :], b3, BIGI)
            p1 = jnp.min(pb, axis=0)
            p = jnp.min(p1, axis=1, keepdims=True)
            bacc = jnp.where(lane32 == t, p, bacc)
            x3 = jnp.where(b3 == p[None, :, :], NEGF, x3)
        bid_ref[...] = bacc


def _final_kernel(c_ref, gi_ref, lab_ref, rnd_ref, v_ref, i_ref, fi_ref):
    x = c_ref[...]
    gi = gi_ref[...]
    x = jnp.where(gi == lab_ref[...], -1e9, x)
    gs = []
    for t in range(20):
        mx = jnp.max(x, axis=1, keepdims=True)
        g = jnp.min(jnp.where(x == mx, gi, BIGI), axis=1, keepdims=True)
        v_ref[:, t:t + 1] = mx
        i_ref[:, t:t + 1] = g
        gs.append(g)
        x = jnp.where(gi == g, NEGF, x)
    rnd = rnd_ref[...]
    i20 = lax.broadcasted_iota(jnp.int32, (CQ, 20), 1)
    rmx = jnp.max(rnd, axis=1, keepdims=True)
    p = jnp.min(jnp.where(rnd == rmx, i20, BIGI), axis=1, keepdims=True)
    fi = gs[0]
    for t in range(1, 20):
        fi = jnp.where(p == t, gs[t], fi)
    fi_ref[...] = fi


def _sc_gather(table, gidx2d, nrows):
    info = plsc.get_sparse_core_info()
    nc, ns = info.num_cores, info.num_subcores
    nw = nc * ns
    per_w = nrows // nw
    nchips = per_w // 128
    mesh = plsc.VectorSubcoreMesh(core_axis_name="c", subcore_axis_name="s")

    @functools.partial(
        pl.kernel,
        out_type=jax.ShapeDtypeStruct((nrows, W), jnp.float32),
        mesh=mesh,
        scratch_types=[
            pltpu.VMEM((8, 128), jnp.int32),
            pltpu.VMEM((per_w, W), jnp.float32),
            pltpu.SemaphoreType.DMA,
        ],
    )
    def k(table_hbm, idx_hbm, out_hbm, idx_v, rows_v, sem):
        wid = lax.axis_index("s") * nc + lax.axis_index("c")
        pltpu.sync_copy(idx_hbm.at[pl.ds(wid * 8, 8)], idx_v)
        copies = [
            pltpu.async_copy(table_hbm.at[idx_v.at[j]],
                             rows_v.at[pl.ds(j * 128, 128)], sem)
            for j in range(nchips)
        ]
        for c in copies:
            c.wait()
        pltpu.sync_copy(rows_v, out_hbm.at[pl.ds(wid * per_w, per_w)])

    return k(table, gidx2d)


def kernel(queries, keys, labels):
    Q = queries.shape[0]
    keys_p = jnp.pad(keys, ((0, NKP - NKEY), (0, 0)))

    sim, bid = pl.pallas_call(
        _sim_kernel,
        grid=(Q // CQ, NCHUNK),
        in_specs=[
            pl.BlockSpec((CQ, 128), lambda i, j: (i, 0)),
            pl.BlockSpec((CK, 128), lambda i, j: (j, 0)),
        ],
        out_specs=[
            pl.BlockSpec((CQ, CK // W, W), lambda i, j: (i, j, 0)),
            pl.BlockSpec((CQ, 32), lambda i, j: (i, 0)),
        ],
        out_shape=[
            jax.ShapeDtypeStruct((Q, NB, W), jnp.float32),
            jax.ShapeDtypeStruct((Q, 32), jnp.int32),
        ],
        scratch_shapes=[pltpu.VMEM((NCHUNK, CQ, 32), jnp.float32)],
    )(queries, keys_p)

    bid24 = bid[:, :TSEL]
    qi = jnp.arange(Q, dtype=jnp.int32)[:, None]
    gidx = (qi * NB + bid24).reshape(32, Q * TSEL // (128 * 32), 128)
    gidx = jnp.pad(gidx, ((0, 0), (0, 8 - gidx.shape[1]), (0, 0)))
    table = sim.reshape(Q * NB, W)
    cand = _sc_gather(table, gidx.reshape(32 * 8, 128), Q * TSEL)

    cand2 = cand.reshape(Q, NCAND)
    gi = (bid24[:, :, None] * W
          + jnp.arange(W, dtype=jnp.int32)).reshape(Q, NCAND)
    rnd = jax.random.uniform(jax.random.key(1), (Q, 20), dtype=jnp.float32)

    vals2, idx2, fi = pl.pallas_call(
        _final_kernel,
        grid=(Q // CQ,),
        in_specs=[
            pl.BlockSpec((CQ, NCAND), lambda i: (i, 0)),
            pl.BlockSpec((CQ, NCAND), lambda i: (i, 0)),
            pl.BlockSpec((CQ, 1), lambda i: (i, 0)),
            pl.BlockSpec((CQ, 20), lambda i: (i, 0)),
        ],
        out_specs=[
            pl.BlockSpec((CQ, 20), lambda i: (i, 0)),
            pl.BlockSpec((CQ, 20), lambda i: (i, 0)),
            pl.BlockSpec((CQ, 1), lambda i: (i, 0)),
        ],
        out_shape=[
            jax.ShapeDtypeStruct((Q, 20), jnp.float32),
            jax.ShapeDtypeStruct((Q, 20), jnp.int32),
            jax.ShapeDtypeStruct((Q, 1), jnp.int32),
        ],
    )(cand2, gi, labels.reshape(Q, 1), rnd)

    return vals2, idx2, fi


# speedup vs baseline: 54.1122x; 1.2843x over previous
import functools

import jax
import jax.numpy as jnp
from jax import lax
from jax.experimental import pallas as pl
from jax.experimental.pallas import tpu as pltpu
from jax.experimental.pallas import tpu_sc as plsc

NKEY = 100000
NKP = 102400
CQ = 256
CK = 4096
NCHUNK = NKP // CK
W = 128
NB = NKP // W
TSEL = 24
NCAND = TSEL * W
NEGF = -1e30
BIGI = 2**30

def _sim_kernel(q_ref, k_ref, sim_ref, bid_ref, macc_ref):
    j = pl.program_id(1)
    sim = lax.dot_general(
        q_ref[...], k_ref[...], (((1,), (1,)), ((), ())),
        preferred_element_type=jnp.float32,
        precision=lax.Precision.DEFAULT,
    )
    col = j * CK + lax.broadcasted_iota(jnp.int32, (CQ, CK), 1)
    sim = jnp.where(col < NKEY, sim, NEGF)
    sim3 = sim.reshape(CQ, CK // W, W)
    sim_ref[...] = sim3
    macc_ref[j] = jnp.max(sim3, axis=2)

    @pl.when(j == NCHUNK - 1)
    def _extract():
        x2 = jnp.concatenate([macc_ref[jj] for jj in range(NCHUNK)], axis=1)
        pos = lax.broadcasted_iota(jnp.int32, (CQ, NB), 1)
        lane32 = lax.broadcasted_iota(jnp.int32, (CQ, 32), 1)
        bacc = jnp.zeros((CQ, 32), jnp.int32)
        for t in range(TSEL):
            mx = jnp.max(x2, axis=1, keepdims=True)
            p = jnp.min(jnp.where(x2 == mx, pos, BIGI), axis=1, keepdims=True)
            bacc = jnp.where(lane32 == t, p, bacc)
            x2 = jnp.where(pos == p, NEGF, x2)
        bid_ref[...] = bacc


def _final_kernel(c_ref, bid_ref, lab_ref, rnd_ref, v_ref, i_ref, fi_ref):
    x = c_ref[...]
    bid = bid_ref[...]
    lane = lax.broadcasted_iota(jnp.int32, (CQ, NCAND), 1)
    slot = lane // W
    gi = jnp.zeros((CQ, NCAND), jnp.int32)
    for s in range(TSEL):
        gi = jnp.where(slot == s, bid[:, s:s + 1], gi)
    gi = gi * W + lane % W
    x = jnp.where(gi == lab_ref[...], -1e9, x)
    gs = []
    for t in range(20):
        mx = jnp.max(x, axis=1, keepdims=True)
        g = jnp.min(jnp.where(x == mx, gi, BIGI), axis=1, keepdims=True)
        v_ref[:, t:t + 1] = mx
        i_ref[:, t:t + 1] = g
        gs.append(g)
        x = jnp.where(gi == g, NEGF, x)
    rnd = rnd_ref[...]
    i20 = lax.broadcasted_iota(jnp.int32, (CQ, 20), 1)
    rmx = jnp.max(rnd, axis=1, keepdims=True)
    p = jnp.min(jnp.where(rnd == rmx, i20, BIGI), axis=1, keepdims=True)
    fi = gs[0]
    for t in range(1, 20):
        fi = jnp.where(p == t, gs[t], fi)
    fi_ref[...] = fi


def _sc_gather(table, gidx2d, nrows):
    info = plsc.get_sparse_core_info()
    nc, ns = info.num_cores, info.num_subcores
    nw = nc * ns
    per_w = nrows // nw
    nchips = per_w // 128
    mesh = plsc.VectorSubcoreMesh(core_axis_name="c", subcore_axis_name="s")

    @functools.partial(
        pl.kernel,
        out_type=jax.ShapeDtypeStruct((nrows, W), jnp.float32),
        mesh=mesh,
        scratch_types=[
            pltpu.VMEM((8, 128), jnp.int32),
            pltpu.VMEM((per_w, W), jnp.float32),
            pltpu.SemaphoreType.DMA,
        ],
    )
    def k(table_hbm, idx_hbm, out_hbm, idx_v, rows_v, sem):
        wid = lax.axis_index("s") * nc + lax.axis_index("c")
        pltpu.sync_copy(idx_hbm.at[pl.ds(wid * 8, 8)], idx_v)
        copies = [
            pltpu.async_copy(table_hbm.at[idx_v.at[j]],
                             rows_v.at[pl.ds(j * 128, 128)], sem)
            for j in range(nchips)
        ]
        for c in copies:
            c.wait()
        pltpu.sync_copy(rows_v, out_hbm.at[pl.ds(wid * per_w, per_w)])

    return k(table, gidx2d)


def kernel(queries, keys, labels):
    Q = queries.shape[0]

    sim, bid = pl.pallas_call(
        _sim_kernel,
        grid=(Q // CQ, NCHUNK),
        in_specs=[
            pl.BlockSpec((CQ, 128), lambda i, j: (i, 0)),
            pl.BlockSpec((CK, 128), lambda i, j: (j, 0)),
        ],
        out_specs=[
            pl.BlockSpec((CQ, CK // W, W), lambda i, j: (i, j, 0)),
            pl.BlockSpec((CQ, 32), lambda i, j: (i, 0)),
        ],
        out_shape=[
            jax.ShapeDtypeStruct((Q, NB, W), jnp.float32),
            jax.ShapeDtypeStruct((Q, 32), jnp.int32),
        ],
        scratch_shapes=[pltpu.VMEM((NCHUNK, CQ, 32), jnp.float32)],
    )(queries, keys)

    bid24 = bid[:, :TSEL]
    qi = jnp.arange(Q, dtype=jnp.int32)[:, None]
    gidx = (qi * NB + bid24).reshape(32, Q * TSEL // (128 * 32), 128)
    gidx = jnp.pad(gidx, ((0, 0), (0, 8 - gidx.shape[1]), (0, 0)))
    table = sim.reshape(Q * NB, W)
    cand = _sc_gather(table, gidx.reshape(32 * 8, 128), Q * TSEL)

    cand2 = cand.reshape(Q, NCAND)
    rnd = jax.random.uniform(jax.random.key(1), (Q, 20), dtype=jnp.float32)

    vals2, idx2, fi = pl.pallas_call(
        _final_kernel,
        grid=(Q // CQ,),
        in_specs=[
            pl.BlockSpec((CQ, NCAND), lambda i: (i, 0)),
            pl.BlockSpec((CQ, 32), lambda i: (i, 0)),
            pl.BlockSpec((CQ, 1), lambda i: (i, 0)),
            pl.BlockSpec((CQ, 20), lambda i: (i, 0)),
        ],
        out_specs=[
            pl.BlockSpec((CQ, 20), lambda i: (i, 0)),
            pl.BlockSpec((CQ, 20), lambda i: (i, 0)),
            pl.BlockSpec((CQ, 1), lambda i: (i, 0)),
        ],
        out_shape=[
            jax.ShapeDtypeStruct((Q, 20), jnp.float32),
            jax.ShapeDtypeStruct((Q, 20), jnp.int32),
            jax.ShapeDtypeStruct((Q, 1), jnp.int32),
        ],
    )(cand2, bid, labels.reshape(Q, 1), rnd)

    return vals2, idx2, fi


# speedup vs baseline: 54.3406x; 1.0042x over previous
import functools

import jax
import jax.numpy as jnp
from jax import lax
from jax.experimental import pallas as pl
from jax.experimental.pallas import tpu as pltpu
from jax.experimental.pallas import tpu_sc as plsc

NKEY = 100000
NKP = 102400
CQ = 256
CK = 4096
NCHUNK = NKP // CK
W = 128
NB = NKP // W
TSEL = 24
NCAND = TSEL * W
NEGF = -1e30
BIGI = 2**30

def _sim_kernel(q_ref, k_ref, sim_ref, bid_ref, macc_ref):
    j = pl.program_id(1)
    sim = lax.dot_general(
        q_ref[...], k_ref[...], (((1,), (1,)), ((), ())),
        preferred_element_type=jnp.float32,
        precision=lax.Precision.DEFAULT,
    )
    @pl.when(j < NCHUNK - 1)
    def _plain():
        sim3 = sim.reshape(CQ, CK // W, W)
        sim_ref[...] = sim3
        macc_ref[j] = jnp.max(sim3, axis=2)

    @pl.when(j == NCHUNK - 1)
    def _extract():
        col = j * CK + lax.broadcasted_iota(jnp.int32, (CQ, CK), 1)
        simm = jnp.where(col < NKEY, sim, NEGF)
        sim3 = simm.reshape(CQ, CK // W, W)
        sim_ref[...] = sim3
        macc_ref[j] = jnp.max(sim3, axis=2)
        x2 = jnp.concatenate([macc_ref[jj] for jj in range(NCHUNK)], axis=1)
        pos = lax.broadcasted_iota(jnp.int32, (CQ, NB), 1)
        lane32 = lax.broadcasted_iota(jnp.int32, (CQ, 32), 1)
        bacc = jnp.zeros((CQ, 32), jnp.int32)
        for t in range(TSEL):
            mx = jnp.max(x2, axis=1, keepdims=True)
            p = jnp.min(jnp.where(x2 == mx, pos, BIGI), axis=1, keepdims=True)
            bacc = jnp.where(lane32 == t, p, bacc)
            x2 = jnp.where(pos == p, NEGF, x2)
        bid_ref[...] = bacc


def _final_kernel(c_ref, bid_ref, lab_ref, rnd_ref, v_ref, i_ref, fi_ref):
    x = c_ref[...]
    bid = bid_ref[...]
    lane = lax.broadcasted_iota(jnp.int32, (CQ, NCAND), 1)
    slot = lane // W
    gi = jnp.zeros((CQ, NCAND), jnp.int32)
    for s in range(TSEL):
        gi = jnp.where(slot == s, bid[:, s:s + 1], gi)
    gi = gi * W + lane % W
    x = jnp.where(gi == lab_ref[...], -1e9, x)
    gs = []
    for t in range(20):
        mx = jnp.max(x, axis=1, keepdims=True)
        g = jnp.min(jnp.where(x == mx, gi, BIGI), axis=1, keepdims=True)
        v_ref[:, t:t + 1] = mx
        i_ref[:, t:t + 1] = g
        gs.append(g)
        x = jnp.where(gi == g, NEGF, x)
    rnd = rnd_ref[...]
    i20 = lax.broadcasted_iota(jnp.int32, (CQ, 20), 1)
    rmx = jnp.max(rnd, axis=1, keepdims=True)
    p = jnp.min(jnp.where(rnd == rmx, i20, BIGI), axis=1, keepdims=True)
    fi = gs[0]
    for t in range(1, 20):
        fi = jnp.where(p == t, gs[t], fi)
    fi_ref[...] = fi


def _sc_gather(table, gidx2d, nrows):
    info = plsc.get_sparse_core_info()
    nc, ns = info.num_cores, info.num_subcores
    nw = nc * ns
    per_w = nrows // nw
    nchips = per_w // 128
    mesh = plsc.VectorSubcoreMesh(core_axis_name="c", subcore_axis_name="s")

    @functools.partial(
        pl.kernel,
        out_type=jax.ShapeDtypeStruct((nrows, W), jnp.float32),
        mesh=mesh,
        scratch_types=[
            pltpu.VMEM((8, 128), jnp.int32),
            pltpu.VMEM((per_w, W), jnp.float32),
            pltpu.SemaphoreType.DMA,
        ],
    )
    def k(table_hbm, idx_hbm, out_hbm, idx_v, rows_v, sem):
        wid = lax.axis_index("s") * nc + lax.axis_index("c")
        pltpu.sync_copy(idx_hbm.at[pl.ds(wid * 8, 8)], idx_v)
        copies = [
            pltpu.async_copy(table_hbm.at[idx_v.at[j]],
                             rows_v.at[pl.ds(j * 128, 128)], sem)
            for j in range(nchips)
        ]
        for c in copies:
            c.wait()
        pltpu.sync_copy(rows_v, out_hbm.at[pl.ds(wid * per_w, per_w)])

    return k(table, gidx2d)


def kernel(queries, keys, labels):
    Q = queries.shape[0]

    sim, bid = pl.pallas_call(
        _sim_kernel,
        grid=(Q // CQ, NCHUNK),
        in_specs=[
            pl.BlockSpec((CQ, 128), lambda i, j: (i, 0)),
            pl.BlockSpec((CK, 128), lambda i, j: (j, 0)),
        ],
        out_specs=[
            pl.BlockSpec((CQ, CK // W, W), lambda i, j: (i, j, 0)),
            pl.BlockSpec((CQ, 32), lambda i, j: (i, 0)),
        ],
        out_shape=[
            jax.ShapeDtypeStruct((Q, NB, W), jnp.float32),
            jax.ShapeDtypeStruct((Q, 32), jnp.int32),
        ],
        scratch_shapes=[pltpu.VMEM((NCHUNK, CQ, 32), jnp.float32)],
    )(queries, keys)

    bid24 = bid[:, :TSEL]
    qi = jnp.arange(Q, dtype=jnp.int32)[:, None]
    gidx = (qi * NB + bid24).reshape(32, Q * TSEL // (128 * 32), 128)
    gidx = jnp.pad(gidx, ((0, 0), (0, 8 - gidx.shape[1]), (0, 0)))
    table = sim.reshape(Q * NB, W)
    cand = _sc_gather(table, gidx.reshape(32 * 8, 128), Q * TSEL)

    cand2 = cand.reshape(Q, NCAND)
    rnd = jax.random.uniform(jax.random.key(1), (Q, 20), dtype=jnp.float32)

    vals2, idx2, fi = pl.pallas_call(
        _final_kernel,
        grid=(Q // CQ,),
        in_specs=[
            pl.BlockSpec((CQ, NCAND), lambda i: (i, 0)),
            pl.BlockSpec((CQ, 32), lambda i: (i, 0)),
            pl.BlockSpec((CQ, 1), lambda i: (i, 0)),
            pl.BlockSpec((CQ, 20), lambda i: (i, 0)),
        ],
        out_specs=[
            pl.BlockSpec((CQ, 20), lambda i: (i, 0)),
            pl.BlockSpec((CQ, 20), lambda i: (i, 0)),
            pl.BlockSpec((CQ, 1), lambda i: (i, 0)),
        ],
        out_shape=[
            jax.ShapeDtypeStruct((Q, 20), jnp.float32),
            jax.ShapeDtypeStruct((Q, 20), jnp.int32),
            jax.ShapeDtypeStruct((Q, 1), jnp.int32),
        ],
    )(cand2, bid, labels.reshape(Q, 1), rnd)

    return vals2, idx2, fi


# speedup vs baseline: 61.8505x; 1.1382x over previous
import functools

import jax
import jax.numpy as jnp
from jax import lax
from jax.experimental import pallas as pl
from jax.experimental.pallas import tpu as pltpu
from jax.experimental.pallas import tpu_sc as plsc

NKEY = 100000
NKP = 102400
CQ = 256
CQA = 512
CK = 4096
NCHUNK = NKP // CK
W = 128
NB = NKP // W
TSEL = 24
NCAND = TSEL * W
NEGF = -1e30
BIGI = 2**30

def _sim_kernel(q_ref, k_ref, sim_ref, bid_ref, macc_ref):
    j = pl.program_id(1)
    sim = lax.dot_general(
        q_ref[...], k_ref[...], (((1,), (1,)), ((), ())),
        preferred_element_type=jnp.float32,
        precision=lax.Precision.DEFAULT,
    )
    @pl.when(j < NCHUNK - 1)
    def _plain():
        sim3 = sim.reshape(CQA, CK // W, W)
        sim_ref[...] = sim3
        macc_ref[j] = jnp.max(sim3, axis=2)

    @pl.when(j == NCHUNK - 1)
    def _extract():
        col = j * CK + lax.broadcasted_iota(jnp.int32, (CQA, CK), 1)
        simm = jnp.where(col < NKEY, sim, NEGF)
        sim3 = simm.reshape(CQA, CK // W, W)
        sim_ref[...] = sim3
        macc_ref[j] = jnp.max(sim3, axis=2)
        x2 = jnp.concatenate([macc_ref[jj] for jj in range(NCHUNK)], axis=1)
        pos = lax.broadcasted_iota(jnp.int32, (CQA, NB), 1)
        lane32 = lax.broadcasted_iota(jnp.int32, (CQA, 32), 1)
        bacc = jnp.zeros((CQA, 32), jnp.int32)
        for t in range(TSEL):
            mx = jnp.max(x2, axis=1, keepdims=True)
            p = jnp.min(jnp.where(x2 == mx, pos, BIGI), axis=1, keepdims=True)
            bacc = jnp.where(lane32 == t, p, bacc)
            x2 = jnp.where(pos == p, NEGF, x2)
        bid_ref[...] = bacc


def _final_kernel(c_ref, bid_ref, lab_ref, rnd_ref, v_ref, i_ref, fi_ref):
    x = c_ref[...]
    bid = bid_ref[...]
    lane = lax.broadcasted_iota(jnp.int32, (CQ, NCAND), 1)
    slot = lane // W
    gi = jnp.zeros((CQ, NCAND), jnp.int32)
    for s in range(TSEL):
        gi = jnp.where(slot == s, bid[:, s:s + 1], gi)
    gi = gi * W + lane % W
    x = jnp.where(gi == lab_ref[...], -1e9, x)
    gs = []
    for t in range(20):
        mx = jnp.max(x, axis=1, keepdims=True)
        g = jnp.min(jnp.where(x == mx, gi, BIGI), axis=1, keepdims=True)
        v_ref[:, t:t + 1] = mx
        i_ref[:, t:t + 1] = g
        gs.append(g)
        x = jnp.where(gi == g, NEGF, x)
    rnd = rnd_ref[...]
    i20 = lax.broadcasted_iota(jnp.int32, (CQ, 20), 1)
    rmx = jnp.max(rnd, axis=1, keepdims=True)
    p = jnp.min(jnp.where(rnd == rmx, i20, BIGI), axis=1, keepdims=True)
    fi = gs[0]
    for t in range(1, 20):
        fi = jnp.where(p == t, gs[t], fi)
    fi_ref[...] = fi


def _sc_gather(table, gidx2d, nrows):
    info = plsc.get_sparse_core_info()
    nc, ns = info.num_cores, info.num_subcores
    nw = nc * ns
    per_w = nrows // nw
    nchips = per_w // 128
    mesh = plsc.VectorSubcoreMesh(core_axis_name="c", subcore_axis_name="s")

    @functools.partial(
        pl.kernel,
        out_type=jax.ShapeDtypeStruct((nrows, W), jnp.float32),
        mesh=mesh,
        scratch_types=[
            pltpu.VMEM((8, 128), jnp.int32),
            pltpu.VMEM((per_w, W), jnp.float32),
            pltpu.SemaphoreType.DMA,
        ],
    )
    def k(table_hbm, idx_hbm, out_hbm, idx_v, rows_v, sem):
        wid = lax.axis_index("s") * nc + lax.axis_index("c")
        pltpu.sync_copy(idx_hbm.at[pl.ds(wid * 8, 8)], idx_v)
        copies = [
            pltpu.async_copy(table_hbm.at[idx_v.at[j]],
                             rows_v.at[pl.ds(j * 128, 128)], sem)
            for j in range(nchips)
        ]
        for c in copies:
            c.wait()
        pltpu.sync_copy(rows_v, out_hbm.at[pl.ds(wid * per_w, per_w)])

    return k(table, gidx2d)


def kernel(queries, keys, labels):
    Q = queries.shape[0]

    sim, bid = pl.pallas_call(
        _sim_kernel,
        grid=(Q // CQA, NCHUNK),
        in_specs=[
            pl.BlockSpec((CQA, 128), lambda i, j: (i, 0)),
            pl.BlockSpec((CK, 128), lambda i, j: (j, 0)),
        ],
        out_specs=[
            pl.BlockSpec((CQA, CK // W, W), lambda i, j: (i, j, 0)),
            pl.BlockSpec((CQA, 32), lambda i, j: (i, 0)),
        ],
        out_shape=[
            jax.ShapeDtypeStruct((Q, NB, W), jnp.float32),
            jax.ShapeDtypeStruct((Q, 32), jnp.int32),
        ],
        scratch_shapes=[pltpu.VMEM((NCHUNK, CQA, 32), jnp.float32)],
    )(queries, keys)

    bid24 = bid[:, :TSEL]
    qi = jnp.arange(Q, dtype=jnp.int32)[:, None]
    gidx = (qi * NB + bid24).reshape(32, Q * TSEL // (128 * 32), 128)
    gidx = jnp.pad(gidx, ((0, 0), (0, 8 - gidx.shape[1]), (0, 0)))
    table = sim.reshape(Q * NB, W)
    cand = _sc_gather(table, gidx.reshape(32 * 8, 128), Q * TSEL)

    cand2 = cand.reshape(Q, NCAND)
    rnd = jax.random.uniform(jax.random.key(1), (Q, 20), dtype=jnp.float32)

    vals2, idx2, fi = pl.pallas_call(
        _final_kernel,
        grid=(Q // CQ,),
        in_specs=[
            pl.BlockSpec((CQ, NCAND), lambda i: (i, 0)),
            pl.BlockSpec((CQ, 32), lambda i: (i, 0)),
            pl.BlockSpec((CQ, 1), lambda i: (i, 0)),
            pl.BlockSpec((CQ, 20), lambda i: (i, 0)),
        ],
        out_specs=[
            pl.BlockSpec((CQ, 20), lambda i: (i, 0)),
            pl.BlockSpec((CQ, 20), lambda i: (i, 0)),
            pl.BlockSpec((CQ, 1), lambda i: (i, 0)),
        ],
        out_shape=[
            jax.ShapeDtypeStruct((Q, 20), jnp.float32),
            jax.ShapeDtypeStruct((Q, 20), jnp.int32),
            jax.ShapeDtypeStruct((Q, 1), jnp.int32),
        ],
    )(cand2, bid, labels.reshape(Q, 1), rnd)

    return vals2, idx2, fi
